# baseline (speedup 1.0000x reference)
import functools

import jax
import jax.numpy as jnp
from jax import lax
from jax.experimental import pallas as pl
from jax.experimental.pallas import tpu as pltpu
from jax.experimental.pallas import tpu_sc as plsc

VOCAB = 1000000
EMB = 64
B = 4096
L = 200
NTOK = B * L
NC = 2
NS = 16
NW = NC * NS

FULLBLK = 7812
TAILW = VOCAB - FULLBLK * 128
KMAX = 245

ROWS_PER_W = NTOK // NW
CHUNK = 256
NCHUNK = ROWS_PER_W // CHUNK
BPL = B // CHUNK

_mesh = plsc.VectorSubcoreMesh(core_axis_name="c", subcore_axis_name="s")
_params = pltpu.CompilerParams(
    use_tc_tiling_on_sc=True, needs_layout_passes=False
)


def _wid():
    return lax.axis_index("s") * NC + lax.axis_index("c")


@functools.partial(
    pl.kernel,
    mesh=_mesh,
    out_type=jax.ShapeDtypeStruct((VOCAB, 128), jnp.float32),
    scratch_types=[
        pltpu.VMEM((EMB, 128), jnp.float32),
        pltpu.VMEM((EMB, 128), jnp.float32),
        pltpu.VMEM((128, 128), jnp.float32),
        pltpu.VMEM((128, 128), jnp.float32),
        pltpu.VMEM((TAILW * EMB,), jnp.float32),
        pltpu.SemaphoreType.DMA,
        pltpu.SemaphoreType.DMA,
        pltpu.SemaphoreType.DMA,
        pltpu.SemaphoreType.DMA,
    ],
    compiler_params=_params,
)
def _transpose_sc(tt_hbm, tail_hbm, scr_hbm, sv0, sv1, dv0, dv1, tl_v,
                  ss0, ss1, ds0, ds1):
    wid = _wid()
    srcs = (sv0, sv1)
    dsts = (dv0, dv1)
    ssems = (ss0, ss1)
    dsems = (ds0, ds1)
    base16 = lax.iota(jnp.int32, 16)

    def load_block(c, s):
        off = pl.multiple_of(c * 128, 128)
        pltpu.async_copy(tt_hbm.at[:, pl.ds(off, 128)], srcs[s], ssems[s])

    def wait_load(s):
        pltpu.make_async_copy(
            tt_hbm.at[:, pl.ds(0, 128)], srcs[s], ssems[s]
        ).wait()

    def store_block(c, s):
        off = pl.multiple_of(c * 128, 128)
        pltpu.async_copy(dsts[s], scr_hbm.at[pl.ds(off, 128)], dsems[s])

    def wait_store(s):
        pltpu.make_async_copy(
            dsts[s], scr_hbm.at[pl.ds(0, 128)], dsems[s]
        ).wait()

    def transpose_block(s):
        def row4(j2, carry):
            for jj in range(4):
                j = j2 * 4 + jj
                jv = base16 * 0 + j
                for x0 in (0, 16, 32, 48):
                    vals = plsc.load_gather(srcs[s], [base16 + x0, jv])
                    dsts[s][j, pl.ds(x0, 16)] = vals
            return carry

        lax.fori_loop(0, 32, row4, 0)

    load_block(wid, 0)

    def body(r, carry):
        for s in range(2):
            k = r * 2 + s
            c = wid + k * NW
            c2 = wid + (k + 1) * NW

            @pl.when(c2 < FULLBLK)
            def _():
                load_block(c2, 1 - s)

            @pl.when(c < FULLBLK)
            def _():
                wait_load(s)

                @pl.when(k >= 2)
                def _():
                    wait_store(s)

                transpose_block(s)
                store_block(c, s)

        return carry

    lax.fori_loop(0, (KMAX + 1) // 2, body, 0)
    wait_store(0)
    wait_store(1)

    @pl.when(wid == FULLBLK % NW)
    def _():
        pltpu.sync_copy(tail_hbm, tl_v)

        def rowt(j, carry):
            for x0 in (0, 16, 32, 48):
                vals = plsc.load_gather(tl_v, [j * EMB + base16 + x0])
                dv0[j, pl.ds(x0, 16)] = vals
            return carry

        lax.fori_loop(0, TAILW, rowt, 0)
        pltpu.sync_copy(
            dv0.at[pl.ds(0, TAILW)], scr_hbm.at[pl.ds(FULLBLK * 128, TAILW)]
        )


@functools.partial(
    pl.kernel,
    mesh=_mesh,
    out_type=jax.ShapeDtypeStruct((L, EMB, B), jnp.float32),
    scratch_types=[
        pltpu.VMEM((CHUNK,), jnp.int32),
        pltpu.VMEM((CHUNK,), jnp.int32),
        pltpu.VMEM((CHUNK, 128), jnp.float32),
        pltpu.VMEM((CHUNK, 128), jnp.float32),
        pltpu.VMEM((1, EMB, CHUNK), jnp.float32),
        pltpu.VMEM((1, EMB, CHUNK), jnp.float32),
        pltpu.SemaphoreType.DMA,
        pltpu.SemaphoreType.DMA,
        pltpu.SemaphoreType.DMA,
        pltpu.SemaphoreType.DMA,
    ],
    compiler_params=_params,
)
def _gather_sc(tokens_hbm, scr_hbm, out_hbm, ix0, ix1, rv0, rv1, tb0, tb1,
               gs0, gs1, os0, os1):
    wid = _wid()
    cbase = wid * NCHUNK
    idxs = (ix0, ix1)
    rows = (rv0, rv1)
    tbufs = (tb0, tb1)
    gsems = (gs0, gs1)
    osems = (os0, os1)
    base16 = lax.iota(jnp.int32, 16)
    groups = [base16 + g * 16 for g in range(CHUNK // 16)]

    def fetch_idx(j, s):
        off = pl.multiple_of((cbase + j) * CHUNK, CHUNK)
        pltpu.sync_copy(tokens_hbm.at[pl.ds(off, CHUNK)], idxs[s])

    def gather(s):
        pltpu.async_copy(scr_hbm.at[idxs[s]], rows[s], gsems[s])

    def wait_gather(s):
        pltpu.make_async_copy(
            scr_hbm.at[pl.ds(0, CHUNK)], rows[s], gsems[s]
        ).wait()

    def transpose_chunk(s):
        def frow(f, carry):
            fv = base16 * 0 + f
            for g in range(CHUNK // 16):
                vals = plsc.load_gather(rows[s], [groups[g], fv])
                tbufs[s][0, f, pl.ds(g * 16, 16)] = vals
            return carry

        lax.fori_loop(0, EMB, frow, 0)

    def put(j, s):
        ch = cbase + j
        l = ch // BPL
        b0 = pl.multiple_of((ch % BPL) * CHUNK, CHUNK)
        pltpu.async_copy(
            tbufs[s],
            out_hbm.at[pl.ds(l, 1), :, pl.ds(b0, CHUNK)],
            osems[s],
        )

    def wait_put(s):
        pltpu.make_async_copy(
            tbufs[s], out_hbm.at[pl.ds(0, 1), :, pl.ds(0, CHUNK)], osems[s]
        ).wait()

    fetch_idx(0, 0)
    gather(0)

    def body(r, carry):
        for s in range(2):
            j = r * 2 + s

            @pl.when(j + 1 < NCHUNK)
            def _():
                fetch_idx(j + 1, 1 - s)
                gather(1 - s)

            wait_gather(s)

            @pl.when(j >= 2)
            def _():
                wait_put(s)

            transpose_chunk(s)
            put(j, s)
        return carry

    lax.fori_loop(0, NCHUNK // 2, body, 0)
    wait_put(0)
    wait_put(1)


def kernel(tokens, table):
    tt = jnp.swapaxes(table, 0, 1)
    tail = table[FULLBLK * 128:].reshape(TAILW * EMB)
    scratch = _transpose_sc(tt, tail)
    flat = jnp.swapaxes(tokens, 0, 1).reshape(NTOK).astype(jnp.int32)
    out = _gather_sc(flat, scratch)
    return jnp.transpose(out, (2, 0, 1))


# speedup vs baseline: 1.8429x; 1.8429x over previous
import functools

import jax
import jax.numpy as jnp
from jax import lax
from jax.experimental import pallas as pl
from jax.experimental.pallas import tpu as pltpu
from jax.experimental.pallas import tpu_sc as plsc

VOCAB = 1000000
EMB = 64
B = 4096
L = 200
NTOK = B * L
NC = 2
NS = 16
NW = NC * NS

FULLBLK = 7812
TAILW = VOCAB - FULLBLK * 128
KMAX = 245

ROWS_PER_W = NTOK // NW
CHUNK = 256
NCHUNK = ROWS_PER_W // CHUNK
BPL = B // CHUNK

_mesh = plsc.VectorSubcoreMesh(core_axis_name="c", subcore_axis_name="s")
_params = pltpu.CompilerParams(
    use_tc_tiling_on_sc=True, needs_layout_passes=False
)


def _wid():
    return lax.axis_index("s") * NC + lax.axis_index("c")


@functools.partial(
    pl.kernel,
    mesh=_mesh,
    out_type=jax.ShapeDtypeStruct((VOCAB, 128), jnp.float32),
    scratch_types=[
        pltpu.VMEM((EMB, 128), jnp.float32),
        pltpu.VMEM((EMB, 128), jnp.float32),
        pltpu.VMEM((128, 128), jnp.float32),
        pltpu.VMEM((128, 128), jnp.float32),
        pltpu.VMEM((TAILW * EMB,), jnp.float32),
        pltpu.SemaphoreType.DMA,
        pltpu.SemaphoreType.DMA,
        pltpu.SemaphoreType.DMA,
        pltpu.SemaphoreType.DMA,
    ],
    compiler_params=_params,
)
def _transpose_sc(tt_hbm, tail_hbm, scr_hbm, sv0, sv1, dv0, dv1, tl_v,
                  ss0, ss1, ds0, ds1):
    wid = _wid()
    srcs = (sv0, sv1)
    dsts = (dv0, dv1)
    ssems = (ss0, ss1)
    dsems = (ds0, ds1)
    base16 = lax.iota(jnp.int32, 16)

    def load_block(c, s):
        off = pl.multiple_of(c * 128, 128)
        pltpu.async_copy(tt_hbm.at[:, pl.ds(off, 128)], srcs[s], ssems[s])

    def wait_load(s):
        pltpu.make_async_copy(
            tt_hbm.at[:, pl.ds(0, 128)], srcs[s], ssems[s]
        ).wait()

    def store_block(c, s):
        off = pl.multiple_of(c * 128, 128)
        pltpu.async_copy(dsts[s], scr_hbm.at[pl.ds(off, 128)], dsems[s])

    def wait_store(s):
        pltpu.make_async_copy(
            dsts[s], scr_hbm.at[pl.ds(0, 128)], dsems[s]
        ).wait()

    def transpose_block(s):
        @plsc.parallel_loop(0, 128, unroll=8)
        def row(j):
            jv = base16 * 0 + j
            for x0 in (0, 16, 32, 48):
                vals = plsc.load_gather(srcs[s], [base16 + x0, jv])
                dsts[s][j, pl.ds(x0, 16)] = vals

    load_block(wid, 0)

    def body(r, carry):
        for s in range(2):
            k = r * 2 + s
            c = wid + k * NW
            c2 = wid + (k + 1) * NW

            @pl.when(c2 < FULLBLK)
            def _():
                load_block(c2, 1 - s)

            @pl.when(c < FULLBLK)
            def _():
                wait_load(s)

                @pl.when(k >= 2)
                def _():
                    wait_store(s)

                transpose_block(s)
                store_block(c, s)

        return carry

    lax.fori_loop(0, (KMAX + 1) // 2, body, 0)
    wait_store(0)
    wait_store(1)

    @pl.when(wid == FULLBLK % NW)
    def _():
        pltpu.sync_copy(tail_hbm, tl_v)

        def rowt(j, carry):
            for x0 in (0, 16, 32, 48):
                vals = plsc.load_gather(tl_v, [j * EMB + base16 + x0])
                dv0[j, pl.ds(x0, 16)] = vals
            return carry

        lax.fori_loop(0, TAILW, rowt, 0)
        pltpu.sync_copy(
            dv0.at[pl.ds(0, TAILW)], scr_hbm.at[pl.ds(FULLBLK * 128, TAILW)]
        )


@functools.partial(
    pl.kernel,
    mesh=_mesh,
    out_type=jax.ShapeDtypeStruct((L, EMB, B), jnp.float32),
    scratch_types=[
        pltpu.VMEM((CHUNK,), jnp.int32),
        pltpu.VMEM((CHUNK,), jnp.int32),
        pltpu.VMEM((CHUNK, 128), jnp.float32),
        pltpu.VMEM((CHUNK, 128), jnp.float32),
        pltpu.VMEM((1, EMB, CHUNK), jnp.float32),
        pltpu.VMEM((1, EMB, CHUNK), jnp.float32),
        pltpu.SemaphoreType.DMA,
        pltpu.SemaphoreType.DMA,
        pltpu.SemaphoreType.DMA,
        pltpu.SemaphoreType.DMA,
    ],
    compiler_params=_params,
)
def _gather_sc(tokens_hbm, scr_hbm, out_hbm, ix0, ix1, rv0, rv1, tb0, tb1,
               gs0, gs1, os0, os1):
    wid = _wid()
    cbase = wid * NCHUNK
    idxs = (ix0, ix1)
    rows = (rv0, rv1)
    tbufs = (tb0, tb1)
    gsems = (gs0, gs1)
    osems = (os0, os1)
    base16 = lax.iota(jnp.int32, 16)
    groups = [base16 + g * 16 for g in range(CHUNK // 16)]

    def fetch_idx(j, s):
        off = pl.multiple_of((cbase + j) * CHUNK, CHUNK)
        pltpu.sync_copy(tokens_hbm.at[pl.ds(off, CHUNK)], idxs[s])

    def gather(s):
        pltpu.async_copy(scr_hbm.at[idxs[s]], rows[s], gsems[s])

    def wait_gather(s):
        pltpu.make_async_copy(
            scr_hbm.at[pl.ds(0, CHUNK)], rows[s], gsems[s]
        ).wait()

    def transpose_chunk(s):
        @plsc.parallel_loop(0, EMB, unroll=2)
        def frow(f):
            fv = base16 * 0 + f
            for g in range(CHUNK // 16):
                vals = plsc.load_gather(rows[s], [groups[g], fv])
                tbufs[s][0, f, pl.ds(g * 16, 16)] = vals

    def put(j, s):
        ch = cbase + j
        l = ch // BPL
        b0 = pl.multiple_of((ch % BPL) * CHUNK, CHUNK)
        pltpu.async_copy(
            tbufs[s],
            out_hbm.at[pl.ds(l, 1), :, pl.ds(b0, CHUNK)],
            osems[s],
        )

    def wait_put(s):
        pltpu.make_async_copy(
            tbufs[s], out_hbm.at[pl.ds(0, 1), :, pl.ds(0, CHUNK)], osems[s]
        ).wait()

    fetch_idx(0, 0)
    gather(0)

    def body(r, carry):
        for s in range(2):
            j = r * 2 + s

            @pl.when(j + 1 < NCHUNK)
            def _():
                fetch_idx(j + 1, 1 - s)
                gather(1 - s)

            wait_gather(s)

            @pl.when(j >= 2)
            def _():
                wait_put(s)

            transpose_chunk(s)
            put(j, s)
        return carry

    lax.fori_loop(0, NCHUNK // 2, body, 0)
    wait_put(0)
    wait_put(1)


def kernel(tokens, table):
    tt = jnp.swapaxes(table, 0, 1)
    tail = table[FULLBLK * 128:].reshape(TAILW * EMB)
    scratch = _transpose_sc(tt, tail)
    flat = jnp.swapaxes(tokens, 0, 1).reshape(NTOK).astype(jnp.int32)
    out = _gather_sc(flat, scratch)
    return jnp.transpose(out, (2, 0, 1))


# speedup vs baseline: 1.9132x; 1.0381x over previous
import functools

import jax
import jax.numpy as jnp
from jax import lax
from jax.experimental import pallas as pl
from jax.experimental.pallas import tpu as pltpu
from jax.experimental.pallas import tpu_sc as plsc

VOCAB = 1000000
EMB = 64
B = 4096
L = 200
NTOK = B * L
NC = 2
NS = 16
NW = NC * NS

FULLBLK = 7812
TAILW = VOCAB - FULLBLK * 128
KMAX = 246

ROWS_PER_W = NTOK // NW
CHUNK = 128
NCHUNK = ROWS_PER_W // CHUNK
BPL = B // CHUNK

_mesh = plsc.VectorSubcoreMesh(core_axis_name="c", subcore_axis_name="s")
_params = pltpu.CompilerParams(
    use_tc_tiling_on_sc=True, needs_layout_passes=False
)


def _wid():
    return lax.axis_index("s") * NC + lax.axis_index("c")


@functools.partial(
    pl.kernel,
    mesh=_mesh,
    out_type=jax.ShapeDtypeStruct((VOCAB, 128), jnp.float32),
    scratch_types=[
        pltpu.VMEM((EMB, 128), jnp.float32),
        pltpu.VMEM((EMB, 128), jnp.float32),
        pltpu.VMEM((EMB, 128), jnp.float32),
        pltpu.VMEM((128, 128), jnp.float32),
        pltpu.VMEM((128, 128), jnp.float32),
        pltpu.VMEM((128, 128), jnp.float32),
        pltpu.VMEM((TAILW * EMB,), jnp.float32),
        pltpu.SemaphoreType.DMA,
        pltpu.SemaphoreType.DMA,
        pltpu.SemaphoreType.DMA,
        pltpu.SemaphoreType.DMA,
        pltpu.SemaphoreType.DMA,
        pltpu.SemaphoreType.DMA,
    ],
    compiler_params=_params,
)
def _transpose_sc(tt_hbm, tail_hbm, scr_hbm, sv0, sv1, sv2, dv0, dv1, dv2,
                  tl_v, ss0, ss1, ss2, ds0, ds1, ds2):
    wid = _wid()
    srcs = (sv0, sv1, sv2)
    dsts = (dv0, dv1, dv2)
    ssems = (ss0, ss1, ss2)
    dsems = (ds0, ds1, ds2)
    base16 = lax.iota(jnp.int32, 16)

    def load_block(c, s):
        off = pl.multiple_of(c * 128, 128)
        pltpu.async_copy(tt_hbm.at[:, pl.ds(off, 128)], srcs[s], ssems[s])

    def wait_load(s):
        pltpu.make_async_copy(
            tt_hbm.at[:, pl.ds(0, 128)], srcs[s], ssems[s]
        ).wait()

    def store_block(c, s):
        off = pl.multiple_of(c * 128, 128)
        pltpu.async_copy(dsts[s], scr_hbm.at[pl.ds(off, 128)], dsems[s])

    def wait_store(s):
        pltpu.make_async_copy(
            dsts[s], scr_hbm.at[pl.ds(0, 128)], dsems[s]
        ).wait()

    def transpose_block(s):
        @plsc.parallel_loop(0, 128, unroll=8)
        def row(j):
            jv = base16 * 0 + j
            for i4 in range(4):
                vals = plsc.load_gather(srcs[s], [base16 + i4 * 16, jv])
                dsts[s][j, pl.ds(i4 * 16, 16)] = vals

    load_block(wid, 0)

    @pl.when(wid + NW < FULLBLK)
    def _():
        load_block(wid + NW, 1)

    def body(r, carry):
        for s in range(3):
            k = r * 3 + s
            c = wid + k * NW
            c2 = wid + (k + 2) * NW

            @pl.when(c2 < FULLBLK)
            def _():
                load_block(c2, (s + 2) % 3)

            @pl.when(c < FULLBLK)
            def _():
                wait_load(s)

                @pl.when(k >= 3)
                def _():
                    wait_store(s)

                transpose_block(s)
                store_block(c, s)

        return carry

    lax.fori_loop(0, KMAX // 3, body, 0)
    wait_store(0)
    wait_store(1)
    wait_store(2)

    @pl.when(wid == FULLBLK % NW)
    def _():
        pltpu.sync_copy(tail_hbm, tl_v)

        def rowt(j, carry):
            for x0 in (0, 16, 32, 48):
                vals = plsc.load_gather(tl_v, [j * EMB + base16 + x0])
                dv0[j, pl.ds(x0, 16)] = vals
            return carry

        lax.fori_loop(0, TAILW, rowt, 0)
        pltpu.sync_copy(
            dv0.at[pl.ds(0, TAILW)], scr_hbm.at[pl.ds(FULLBLK * 128, TAILW)]
        )


@functools.partial(
    pl.kernel,
    mesh=_mesh,
    out_type=jax.ShapeDtypeStruct((L, EMB, B), jnp.float32),
    scratch_types=[
        pltpu.VMEM((ROWS_PER_W,), jnp.int32),
        pltpu.VMEM((CHUNK, 128), jnp.float32),
        pltpu.VMEM((CHUNK, 128), jnp.float32),
        pltpu.VMEM((CHUNK, 128), jnp.float32),
        pltpu.VMEM((CHUNK, 128), jnp.float32),
        pltpu.VMEM((1, EMB, CHUNK), jnp.float32),
        pltpu.VMEM((1, EMB, CHUNK), jnp.float32),
        pltpu.SemaphoreType.DMA,
        pltpu.SemaphoreType.DMA,
        pltpu.SemaphoreType.DMA,
        pltpu.SemaphoreType.DMA,
        pltpu.SemaphoreType.DMA,
        pltpu.SemaphoreType.DMA,
    ],
    compiler_params=_params,
)
def _gather_sc(tokens_hbm, scr_hbm, out_hbm, idx_all, rv0, rv1, rv2, rv3,
               tb0, tb1, gs0, gs1, gs2, gs3, os0, os1):
    wid = _wid()
    cbase = wid * NCHUNK
    rows = (rv0, rv1, rv2, rv3)
    tbufs = (tb0, tb1)
    gsems = (gs0, gs1, gs2, gs3)
    osems = (os0, os1)
    base16 = lax.iota(jnp.int32, 16)
    groups = [base16 + g * 16 for g in range(CHUNK // 16)]

    pltpu.sync_copy(
        tokens_hbm.at[pl.ds(pl.multiple_of(cbase * CHUNK, CHUNK), ROWS_PER_W)],
        idx_all,
    )

    def gather(j, s):
        pltpu.async_copy(
            scr_hbm.at[idx_all.at[pl.ds(j * CHUNK, CHUNK)]], rows[s], gsems[s]
        )

    def wait_gather(s):
        pltpu.make_async_copy(
            scr_hbm.at[pl.ds(0, CHUNK)], rows[s], gsems[s]
        ).wait()

    def transpose_chunk(sg, st):
        @plsc.parallel_loop(0, EMB, unroll=4)
        def frow(f):
            fv = base16 * 0 + f
            for g in range(CHUNK // 16):
                vals = plsc.load_gather(rows[sg], [groups[g], fv])
                tbufs[st][0, f, pl.ds(g * 16, 16)] = vals

    def put(j, st):
        ch = cbase + j
        l = ch // BPL
        b0 = pl.multiple_of((ch % BPL) * CHUNK, CHUNK)
        pltpu.async_copy(
            tbufs[st],
            out_hbm.at[pl.ds(l, 1), :, pl.ds(b0, CHUNK)],
            osems[st],
        )

    def wait_put(st):
        pltpu.make_async_copy(
            tbufs[st], out_hbm.at[pl.ds(0, 1), :, pl.ds(0, CHUNK)], osems[st]
        ).wait()

    gather(0, 0)
    gather(1, 1)
    gather(2, 2)

    def body(r, carry):
        for u in range(4):
            j = r * 4 + u
            sg = u
            st = u % 2

            @pl.when(j + 3 < NCHUNK)
            def _():
                gather(j + 3, (u + 3) % 4)

            wait_gather(sg)

            @pl.when(j >= 2)
            def _():
                wait_put(st)

            transpose_chunk(sg, st)
            put(j, st)
        return carry

    lax.fori_loop(0, NCHUNK // 4, body, 0)
    wait_put(0)
    wait_put(1)


def kernel(tokens, table):
    tt = jnp.swapaxes(table, 0, 1)
    tail = table[FULLBLK * 128:].reshape(TAILW * EMB)
    scratch = _transpose_sc(tt, tail)
    flat = jnp.swapaxes(tokens, 0, 1).reshape(NTOK).astype(jnp.int32)
    out = _gather_sc(flat, scratch)
    return jnp.transpose(out, (2, 0, 1))
